# Initial kernel scaffold; baseline (speedup 1.0000x reference)
import jax
import jax.numpy as jnp
from jax.experimental import pallas as pl


def kernel(x, proj_W, proj_b, att_src_b, att_dst_b, att_src_u, att_dst_u, k_lin_W, k_lin_b, q, lin_W, lin_b, edge_index_boundary, edge_index_upper):
    raise NotImplementedError("write your pallas kernel here")


# speedup vs baseline: 12.7499x; 12.7499x over previous
import functools
import jax
import jax.numpy as jnp
import numpy as np
from jax.experimental import pallas as pl

N = 10000
E = 320000
D_IN = 128
HID = 128
HEADS = 8
D_HEAD = HID // HEADS
OUT = 2

_BLK = 1000
_GRID = N // _BLK


def _k1_body(x_ref, W_ref, b_ref, Ab_ref, Au_ref, h_ref, ab_ref, au_ref):
    h = jnp.dot(x_ref[...], W_ref[...], preferred_element_type=jnp.float32)
    h = h + b_ref[...]
    h_ref[...] = h
    ab_ref[...] = jnp.dot(h, Ab_ref[...], preferred_element_type=jnp.float32)
    au_ref[...] = jnp.dot(h, Au_ref[...], preferred_element_type=jnp.float32)


def _k1(x, W, b2, Ab, Au):
    return pl.pallas_call(
        _k1_body,
        grid=(_GRID,),
        in_specs=[
            pl.BlockSpec((_BLK, D_IN), lambda i: (i, 0)),
            pl.BlockSpec((D_IN, HID), lambda i: (0, 0)),
            pl.BlockSpec((1, HID), lambda i: (0, 0)),
            pl.BlockSpec((HID, 2 * HEADS), lambda i: (0, 0)),
            pl.BlockSpec((HID, 2 * HEADS), lambda i: (0, 0)),
        ],
        out_specs=[
            pl.BlockSpec((_BLK, HID), lambda i: (i, 0)),
            pl.BlockSpec((_BLK, 2 * HEADS), lambda i: (i, 0)),
            pl.BlockSpec((_BLK, 2 * HEADS), lambda i: (i, 0)),
        ],
        out_shape=[
            jax.ShapeDtypeStruct((N, HID), jnp.float32),
            jax.ShapeDtypeStruct((N, 2 * HEADS), jnp.float32),
            jax.ShapeDtypeStruct((N, 2 * HEADS), jnp.float32),
        ],
    )(x, W, b2, Ab, Au)


def _k3_body(numb_ref, denb_ref, numu_ref, denu_ref, W_ref, bk_ref, E_ref,
             acc_ref):
    i = pl.program_id(0)

    @pl.when(i == 0)
    def _():
        acc_ref[...] = jnp.zeros_like(acc_ref)

    def one(num_ref, den_ref):
        num = num_ref[0] + num_ref[1]
        den = den_ref[0] + den_ref[1]
        dr = jnp.dot(1.0 / (den + 1e-16), E_ref[...],
                     preferred_element_type=jnp.float32)
        o = jnp.maximum(num * dr, 0.0)
        s = jnp.sum(o, axis=0)
        t = jnp.sum(jnp.tanh(
            jnp.dot(o, W_ref[...], preferred_element_type=jnp.float32)
            + bk_ref[...]), axis=0)
        return s, t

    sb, tb = one(numb_ref, denb_ref)
    su, tu = one(numu_ref, denu_ref)
    z = jnp.zeros((4, HID), jnp.float32)
    upd = jnp.concatenate([sb[None], su[None], tb[None], tu[None], z], axis=0)
    acc_ref[...] += upd


def _k3(numb, denb, numu, denu, kW, kb2, Emat):
    return pl.pallas_call(
        _k3_body,
        grid=(_GRID,),
        in_specs=[
            pl.BlockSpec((2, _BLK, HID), lambda i: (0, i, 0)),
            pl.BlockSpec((2, _BLK, HEADS), lambda i: (0, i, 0)),
            pl.BlockSpec((2, _BLK, HID), lambda i: (0, i, 0)),
            pl.BlockSpec((2, _BLK, HEADS), lambda i: (0, i, 0)),
            pl.BlockSpec((HID, HID), lambda i: (0, 0)),
            pl.BlockSpec((1, HID), lambda i: (0, 0)),
            pl.BlockSpec((HEADS, HID), lambda i: (0, 0)),
        ],
        out_specs=pl.BlockSpec((8, HID), lambda i: (0, 0)),
        out_shape=jax.ShapeDtypeStruct((8, HID), jnp.float32),
    )(numb, denb, numu, denu, kW, kb2, Emat)


def _edge_stage_jax(h, ab, au, ei_b, ei_u):
    outs = []
    for a, ei in ((ab, ei_b), (au, ei_u)):
        src, dst = ei[0], ei[1]
        alpha = a[:, :HEADS][src] + a[:, HEADS:][dst]
        alpha = jnp.where(alpha >= 0, alpha, 0.2 * alpha)
        ex = jnp.exp(alpha)
        den = jax.ops.segment_sum(ex, dst, num_segments=N)
        msg = h[src] * jnp.repeat(ex, D_HEAD, axis=1)
        num = jax.ops.segment_sum(msg, dst, num_segments=N)
        z128 = jnp.zeros((1, N, HID), jnp.float32)
        z8 = jnp.zeros((1, N, HEADS), jnp.float32)
        outs.append((jnp.concatenate([num[None], z128], axis=0),
                     jnp.concatenate([den[None], z8], axis=0)))
    return outs


def kernel(x, proj_W, proj_b, att_src_b, att_dst_b, att_src_u, att_dst_u,
           k_lin_W, k_lin_b, q, lin_W, lin_b,
           edge_index_boundary, edge_index_upper):
    eye = jnp.eye(HEADS, dtype=jnp.float32)

    def amat(a_src, a_dst):
        ms = (a_src[:, :, None] * eye[:, None, :]).reshape(HID, HEADS)
        md = (a_dst[:, :, None] * eye[:, None, :]).reshape(HID, HEADS)
        return jnp.concatenate([ms, md], axis=1)

    Ab = amat(att_src_b, att_dst_b)
    Au = amat(att_src_u, att_dst_u)
    h, ab, au = _k1(x, proj_W, proj_b[None, :], Ab, Au)

    (numb, denb), (numu, denu) = _edge_stage_jax(
        h, ab, au, edge_index_boundary, edge_index_upper)

    Emat = jnp.repeat(eye, D_HEAD, axis=1)
    acc = _k3(numb, denb, numu, denu, k_lin_W, k_lin_b[None, :], Emat)

    sb, su, tb, tu = acc[0], acc[1], acc[2], acc[3]
    k = jnp.stack([tb, tu]) / N
    score = (q[None, :] * k).sum(-1)
    attn = jax.nn.softmax(score, axis=0)
    pooled = attn[0] * sb + attn[1] * su
    out = pooled[None, :] @ lin_W + lin_b[None, :]
    return jax.nn.sigmoid(out)


# speedup vs baseline: 28.0107x; 2.1969x over previous
import functools
import jax
import jax.numpy as jnp
import numpy as np
from jax import lax
from jax.experimental import pallas as pl
from jax.experimental.pallas import tpu as pltpu
from jax.experimental.pallas import tpu_sc as plsc

N = 10000
E = 320000
D_IN = 128
HID = 128
HEADS = 8
D_HEAD = HID // HEADS
OUT = 2

_BLK = 1000
_GRID = N // _BLK

_NW = 16
_C = 64
_EP = ((E + _NW * _C - 1) // (_NW * _C)) * (_NW * _C)
_EPW = _EP // _NW
_NCH = _EPW // _C
_NP = 10240
_RPT = _NP // 16
_NPD = _NP // 8
_RPD = _NPD // 16


def _k1_body(x_ref, W_ref, b_ref, A_ref, h_ref, t_ref):
    h = jnp.dot(x_ref[...], W_ref[...], preferred_element_type=jnp.float32)
    h = h + b_ref[...]
    h_ref[...] = h
    t_ref[...] = jnp.dot(h, A_ref[...], preferred_element_type=jnp.float32)


def _k1(x, W, b2, A):
    return pl.pallas_call(
        _k1_body,
        grid=(_GRID,),
        in_specs=[
            pl.BlockSpec((_BLK, D_IN), lambda i: (i, 0)),
            pl.BlockSpec((D_IN, HID), lambda i: (0, 0)),
            pl.BlockSpec((1, HID), lambda i: (0, 0)),
            pl.BlockSpec((HID, HID), lambda i: (0, 0)),
        ],
        out_specs=[
            pl.BlockSpec((_BLK, HID), lambda i: (i, 0)),
            pl.BlockSpec((_BLK, HID), lambda i: (i, 0)),
        ],
        out_shape=[
            jax.ShapeDtypeStruct((N, HID), jnp.float32),
            jax.ShapeDtypeStruct((N, HID), jnp.float32),
        ],
    )(x, W, b2, A)


def _k3_body(numb_ref, denb_ref, numu_ref, denu_ref, W_ref, bk_ref, E_ref,
             acc_ref):
    i = pl.program_id(0)

    @pl.when(i == 0)
    def _():
        acc_ref[...] = jnp.zeros_like(acc_ref)

    def one(num_ref, den_ref, o):
        num = num_ref[0]
        den = den_ref[0, :, o:o + HEADS]
        dr = jnp.dot(1.0 / (den + 1e-16), E_ref[...],
                     preferred_element_type=jnp.float32)
        o = jnp.maximum(num * dr, 0.0)
        s = jnp.sum(o, axis=0)
        t = jnp.sum(jnp.tanh(
            jnp.dot(o, W_ref[...], preferred_element_type=jnp.float32)
            + bk_ref[...]), axis=0)
        return s, t

    sb, tb = one(numb_ref, denb_ref, 0)
    su, tu = one(numu_ref, denu_ref, HEADS)
    z = jnp.zeros((4, HID), jnp.float32)
    upd = jnp.concatenate([sb[None], su[None], tb[None], tu[None], z], axis=0)
    acc_ref[...] += upd


def _k3(num_all, den_all, kW, kb2, Emat):
    return pl.pallas_call(
        _k3_body,
        grid=(_GRID,),
        in_specs=[
            pl.BlockSpec((1, _BLK, HID), lambda i: (0, i, 0)),
            pl.BlockSpec((1, _BLK, 2 * HEADS), lambda i: (0, i, 0)),
            pl.BlockSpec((1, _BLK, HID), lambda i: (1, i, 0)),
            pl.BlockSpec((1, _BLK, 2 * HEADS), lambda i: (1, i, 0)),
            pl.BlockSpec((HID, HID), lambda i: (0, 0)),
            pl.BlockSpec((1, HID), lambda i: (0, 0)),
            pl.BlockSpec((HEADS, HID), lambda i: (0, 0)),
        ],
        out_specs=pl.BlockSpec((8, HID), lambda i: (0, 0)),
        out_shape=jax.ShapeDtypeStruct((8, HID), jnp.float32),
    )(num_all, den_all, num_all, den_all, kW, kb2, Emat)


def _sc_body(h_hbm, t_hbm, src_all, dst_all,
             num_all, den_all,
             src_i, dst_i, srcC, dstA, dstC, S, Dv, exP, hbuf,
             num_s, den_s, semA, semB, semH):
    sid = lax.axis_index("s")
    wid = sid
    row0 = sid * _RPT
    zero16i = jnp.zeros((16,), jnp.int32)
    zero16f = jnp.zeros((16,), jnp.float32)

    def zp(i, c):
        exP[i >> 3, pl.ds((i & 7) * 16, 16)] = zero16f
        return c
    lax.fori_loop(0, _C * 8, zp, 0)

    def relbody(rel, cr):
        for j in range(_RPT // _C):
            pltpu.sync_copy(exP, num_s.at[pl.ds(row0 + j * _C, _C)])
        pltpu.sync_copy(exP, den_s.at[pl.ds(sid * _RPD, _C)])
        pltpu.sync_copy(exP.at[pl.ds(0, _RPD - _C)],
                        den_s.at[pl.ds(sid * _RPD + _C, _RPD - _C)])
        plsc.subcore_barrier()

        def chunk(ch, c):
            base = pl.multiple_of(rel * _EP + wid * _EPW + ch * _C, _C)
            pltpu.sync_copy(src_all.at[pl.ds(base, _C)], src_i)
            pltpu.sync_copy(dst_all.at[pl.ds(base, _C)], dst_i)
            for j in range(_C // 16):
                sv = src_i[pl.ds(j * 16, 16)]
                dv = dst_i[pl.ds(j * 16, 16)]
                srcC[pl.ds(j * 16, 16)] = jnp.minimum(sv, N - 1)
                dstA[pl.ds(j * 16, 16)] = jnp.minimum(dv, N - 1)
                dc = jnp.minimum(dv, N)
                dstC[pl.ds(j * 16, 16)] = dc
                dst_i[pl.ds(j * 16, 16)] = dc >> 3
            gA = pltpu.async_copy(t_hbm.at[srcC], S, semA)
            gD = pltpu.async_copy(t_hbm.at[dstA], Dv, semB)
            gH = pltpu.async_copy(h_hbm.at[srcC], hbuf, semH)
            gA.wait()
            gD.wait()

            def exloop(j, c2):
                dvv = dstC[pl.ds(j * 16, 16)]
                for l in range(16):
                    e = j * 16 + l
                    v = S[e, pl.ds(0, 16)] + Dv[e, pl.ds(16, 16)]
                    v = jnp.where(v >= 0, v, 0.2 * v)
                    ex = jnp.exp(v)
                    S[e, pl.ds(0, 16)] = ex
                    cb = (dvv[l] & 7) * 16
                    exP[e, pl.ds(cb, 16)] = ex
                return c2
            lax.fori_loop(0, _C // 16, exloop, 0)

            pltpu.sync_copy(exP, den_s.at[dst_i], add=True)

            def clr(j, c2):
                dvv = dstC[pl.ds(j * 16, 16)]
                for l in range(16):
                    cb = (dvv[l] & 7) * 16
                    exP[j * 16 + l, pl.ds(cb, 16)] = zero16f
                return c2
            lax.fori_loop(0, _C // 16, clr, 0)

            gH.wait()

            def mb(e, c2):
                wv = S[e, pl.ds(0, 16)]
                for hh in range(HEADS):
                    w = lax.gather(
                        wv, (zero16i + (rel * HEADS + hh))[:, None],
                        lax.GatherDimensionNumbers(
                            offset_dims=(), collapsed_slice_dims=(0,),
                            start_index_map=(0,)),
                        slice_sizes=(1,),
                        mode=lax.GatherScatterMode.PROMISE_IN_BOUNDS)
                    hv = hbuf[e, pl.ds(hh * 16, 16)]
                    hbuf[e, pl.ds(hh * 16, 16)] = hv * w
                return c2
            lax.fori_loop(0, _C, mb, 0)

            pltpu.sync_copy(hbuf, num_s.at[dstC], add=True)
            return c

        lax.fori_loop(0, _NCH, chunk, 0)
        plsc.subcore_barrier()

        pltpu.sync_copy(num_s.at[pl.ds(row0, _RPT)],
                        num_all.at[rel, pl.ds(row0, _RPT)])
        pltpu.sync_copy(den_s.at[pl.ds(sid * _RPD, _RPD)],
                        den_all.at[rel, pl.ds(sid * _RPD, _RPD)])
        plsc.subcore_barrier()
        return cr

    lax.fori_loop(0, 2, relbody, 0)


def _edge_stage_sc(h, t, ei_b, ei_u):
    pad = jnp.full((_EP - E,), N, jnp.int32)
    src_all = jnp.concatenate([ei_b[0], pad, ei_u[0], pad])
    dst_all = jnp.concatenate([ei_b[1], pad, ei_u[1], pad])

    mesh = plsc.VectorSubcoreMesh(core_axis_name="c", subcore_axis_name="s",
                                  num_cores=1)
    f = pl.kernel(
        _sc_body,
        out_type=[
            jax.ShapeDtypeStruct((2, _NP, HID), jnp.float32),
            jax.ShapeDtypeStruct((2, _NPD, HID), jnp.float32),
        ],
        mesh=mesh,
        scratch_types=[
            pltpu.VMEM((_C,), jnp.int32),
            pltpu.VMEM((_C,), jnp.int32),
            pltpu.VMEM((_C,), jnp.int32),
            pltpu.VMEM((_C,), jnp.int32),
            pltpu.VMEM((_C,), jnp.int32),
            pltpu.VMEM((_C, HID), jnp.float32),
            pltpu.VMEM((_C, HID), jnp.float32),
            pltpu.VMEM((_C, HID), jnp.float32),
            pltpu.VMEM((_C, HID), jnp.float32),
            pltpu.VMEM_SHARED((_NP, HID), jnp.float32),
            pltpu.VMEM_SHARED((_NPD, HID), jnp.float32),
            pltpu.SemaphoreType.DMA,
            pltpu.SemaphoreType.DMA,
            pltpu.SemaphoreType.DMA,
        ],
    )
    num_all, den_all = f(h, t, src_all, dst_all)
    den_all = den_all.reshape(2, _NP, 2 * HEADS)
    return num_all, den_all


def kernel(x, proj_W, proj_b, att_src_b, att_dst_b, att_src_u, att_dst_u,
           k_lin_W, k_lin_b, q, lin_W, lin_b,
           edge_index_boundary, edge_index_upper):
    eye = jnp.eye(HEADS, dtype=jnp.float32)

    def amat(a_src, a_dst):
        ms = (a_src[:, :, None] * eye[:, None, :]).reshape(HID, HEADS)
        md = (a_dst[:, :, None] * eye[:, None, :]).reshape(HID, HEADS)
        return jnp.concatenate([ms, md], axis=1)

    Ab = amat(att_src_b, att_dst_b)
    Au = amat(att_src_u, att_dst_u)
    A = jnp.concatenate(
        [Ab[:, :HEADS], Au[:, :HEADS], Ab[:, HEADS:], Au[:, HEADS:],
         jnp.zeros((HID, HID - 4 * HEADS), jnp.float32)], axis=1)
    h, t = _k1(x, proj_W, proj_b[None, :], A)

    num_all, den_all = _edge_stage_sc(
        h, t, edge_index_boundary, edge_index_upper)

    Emat = jnp.repeat(eye, D_HEAD, axis=1)
    acc = _k3(num_all, den_all, k_lin_W, k_lin_b[None, :], Emat)

    sb, su, tb, tu = acc[0], acc[1], acc[2], acc[3]
    k = jnp.stack([tb, tu]) / N
    score = (q[None, :] * k).sum(-1)
    attn = jax.nn.softmax(score, axis=0)
    pooled = attn[0] * sb + attn[1] * su
    out = pooled[None, :] @ lin_W + lin_b[None, :]
    return jax.nn.sigmoid(out)


# speedup vs baseline: 44.7539x; 1.5977x over previous
import functools
import jax
import jax.numpy as jnp
import numpy as np
from jax import lax
from jax.experimental import pallas as pl
from jax.experimental.pallas import tpu as pltpu
from jax.experimental.pallas import tpu_sc as plsc

N = 10000
E = 320000
D_IN = 128
HID = 128
HEADS = 8
D_HEAD = HID // HEADS
OUT = 2

_BLK = 1000
_GRID = N // _BLK

_NW = 32
_C = 64
_EP = ((E + _NW * _C - 1) // (_NW * _C)) * (_NW * _C)
_EPW = _EP // _NW
_NCH = _EPW // _C
_NP = 10240
_RPT = _NP // 16
_NPD = _NP // 8
_RPD = _NPD // 16


def _k1_body(x_ref, W_ref, b_ref, A_ref, h_ref, t_ref):
    h = jnp.dot(x_ref[...], W_ref[...], preferred_element_type=jnp.float32)
    h = h + b_ref[...]
    h_ref[...] = h
    t_ref[...] = jnp.dot(h, A_ref[...], preferred_element_type=jnp.float32)


def _k1(x, W, b2, A):
    return pl.pallas_call(
        _k1_body,
        grid=(_GRID,),
        in_specs=[
            pl.BlockSpec((_BLK, D_IN), lambda i: (i, 0)),
            pl.BlockSpec((D_IN, HID), lambda i: (0, 0)),
            pl.BlockSpec((1, HID), lambda i: (0, 0)),
            pl.BlockSpec((HID, HID), lambda i: (0, 0)),
        ],
        out_specs=[
            pl.BlockSpec((_BLK, HID), lambda i: (i, 0)),
            pl.BlockSpec((_BLK, HID), lambda i: (i, 0)),
        ],
        out_shape=[
            jax.ShapeDtypeStruct((N, HID), jnp.float32),
            jax.ShapeDtypeStruct((N, HID), jnp.float32),
        ],
    )(x, W, b2, A)


def _k3_body(numb_ref, denb_ref, numu_ref, denu_ref, W_ref, bk_ref, E_ref,
             acc_ref):
    i = pl.program_id(0)

    @pl.when(i == 0)
    def _():
        acc_ref[...] = jnp.zeros_like(acc_ref)

    def one(num_ref, den_ref, o):
        num = num_ref[0, 0] + num_ref[0, 1]
        den = (den_ref[0, 0, :, o:o + HEADS]
               + den_ref[0, 1, :, o:o + HEADS])
        dr = jnp.dot(1.0 / (den + 1e-16), E_ref[...],
                     preferred_element_type=jnp.float32)
        o = jnp.maximum(num * dr, 0.0)
        s = jnp.sum(o, axis=0)
        t = jnp.sum(jnp.tanh(
            jnp.dot(o, W_ref[...], preferred_element_type=jnp.float32)
            + bk_ref[...]), axis=0)
        return s, t

    sb, tb = one(numb_ref, denb_ref, 0)
    su, tu = one(numu_ref, denu_ref, HEADS)
    z = jnp.zeros((4, HID), jnp.float32)
    upd = jnp.concatenate([sb[None], su[None], tb[None], tu[None], z], axis=0)
    acc_ref[...] += upd


def _k3(num_all, den_all, kW, kb2, Emat):
    return pl.pallas_call(
        _k3_body,
        grid=(_GRID,),
        in_specs=[
            pl.BlockSpec((1, 2, _BLK, HID), lambda i: (0, 0, i, 0)),
            pl.BlockSpec((1, 2, _BLK, 2 * HEADS), lambda i: (0, 0, i, 0)),
            pl.BlockSpec((1, 2, _BLK, HID), lambda i: (1, 0, i, 0)),
            pl.BlockSpec((1, 2, _BLK, 2 * HEADS), lambda i: (1, 0, i, 0)),
            pl.BlockSpec((HID, HID), lambda i: (0, 0)),
            pl.BlockSpec((1, HID), lambda i: (0, 0)),
            pl.BlockSpec((HEADS, HID), lambda i: (0, 0)),
        ],
        out_specs=pl.BlockSpec((8, HID), lambda i: (0, 0)),
        out_shape=jax.ShapeDtypeStruct((8, HID), jnp.float32),
    )(num_all, den_all, num_all, den_all, kW, kb2, Emat)


def _sc_body(h_hbm, t_hbm, src_all, dst_all,
             num_all, den_all,
             src_i, dst_i, srcC, dstA, dstC, S, Dv, exP, hbuf,
             num_s, den_s, semA, semB, semH):
    cid = lax.axis_index("c")
    sid = lax.axis_index("s")
    wid = sid * 2 + cid
    row0 = sid * _RPT
    zero16i = jnp.zeros((16,), jnp.int32)
    zero16f = jnp.zeros((16,), jnp.float32)

    def zp(i, c):
        exP[i >> 3, pl.ds((i & 7) * 16, 16)] = zero16f
        return c
    lax.fori_loop(0, _C * 8, zp, 0)

    def relbody(rel, cr):
        for j in range(_RPT // _C):
            pltpu.sync_copy(exP, num_s.at[pl.ds(row0 + j * _C, _C)])
        pltpu.sync_copy(exP, den_s.at[pl.ds(sid * _RPD, _C)])
        pltpu.sync_copy(exP.at[pl.ds(0, _RPD - _C)],
                        den_s.at[pl.ds(sid * _RPD + _C, _RPD - _C)])
        plsc.subcore_barrier()

        def chunk(ch, c):
            base = pl.multiple_of(rel * _EP + wid * _EPW + ch * _C, _C)
            pltpu.sync_copy(src_all.at[pl.ds(base, _C)], src_i)
            pltpu.sync_copy(dst_all.at[pl.ds(base, _C)], dst_i)
            for j in range(_C // 16):
                sv = src_i[pl.ds(j * 16, 16)]
                dv = dst_i[pl.ds(j * 16, 16)]
                srcC[pl.ds(j * 16, 16)] = jnp.minimum(sv, N - 1)
                dstA[pl.ds(j * 16, 16)] = jnp.minimum(dv, N - 1)
                dc = jnp.minimum(dv, N)
                dstC[pl.ds(j * 16, 16)] = dc
                dst_i[pl.ds(j * 16, 16)] = dc >> 3
            gA = pltpu.async_copy(t_hbm.at[srcC], S, semA)
            gD = pltpu.async_copy(t_hbm.at[dstA], Dv, semB)
            gH = pltpu.async_copy(h_hbm.at[srcC], hbuf, semH)
            gA.wait()
            gD.wait()

            def exloop(j, c2):
                dvv = dstC[pl.ds(j * 16, 16)]
                for l in range(16):
                    e = j * 16 + l
                    v = S[e, pl.ds(0, 16)] + Dv[e, pl.ds(16, 16)]
                    v = jnp.where(v >= 0, v, 0.2 * v)
                    ex = jnp.exp(v)
                    S[e, pl.ds(0, 16)] = ex
                    cb = (dvv[l] & 7) * 16
                    exP[e, pl.ds(cb, 16)] = ex
                return c2
            lax.fori_loop(0, _C // 16, exloop, 0)

            pltpu.sync_copy(exP, den_s.at[dst_i], add=True)

            def clr(j, c2):
                dvv = dstC[pl.ds(j * 16, 16)]
                for l in range(16):
                    cb = (dvv[l] & 7) * 16
                    exP[j * 16 + l, pl.ds(cb, 16)] = zero16f
                return c2
            lax.fori_loop(0, _C // 16, clr, 0)

            gH.wait()

            def mb(e, c2):
                wv = S[e, pl.ds(0, 16)]
                for hh in range(HEADS):
                    w = lax.gather(
                        wv, (zero16i + (rel * HEADS + hh))[:, None],
                        lax.GatherDimensionNumbers(
                            offset_dims=(), collapsed_slice_dims=(0,),
                            start_index_map=(0,)),
                        slice_sizes=(1,),
                        mode=lax.GatherScatterMode.PROMISE_IN_BOUNDS)
                    hv = hbuf[e, pl.ds(hh * 16, 16)]
                    hbuf[e, pl.ds(hh * 16, 16)] = hv * w
                return c2
            lax.fori_loop(0, _C, mb, 0)

            pltpu.sync_copy(hbuf, num_s.at[dstC], add=True)
            return c

        lax.fori_loop(0, _NCH, chunk, 0)
        plsc.subcore_barrier()

        pltpu.sync_copy(num_s.at[pl.ds(row0, _RPT)],
                        num_all.at[rel, cid, pl.ds(row0, _RPT)])
        pltpu.sync_copy(den_s.at[pl.ds(sid * _RPD, _RPD)],
                        den_all.at[rel, cid, pl.ds(sid * _RPD, _RPD)])
        plsc.subcore_barrier()
        return cr

    lax.fori_loop(0, 2, relbody, 0)


def _edge_stage_sc(h, t, ei_b, ei_u):
    pad = jnp.full((_EP - E,), N, jnp.int32)
    src_all = jnp.concatenate([ei_b[0], pad, ei_u[0], pad])
    dst_all = jnp.concatenate([ei_b[1], pad, ei_u[1], pad])

    mesh = plsc.VectorSubcoreMesh(core_axis_name="c", subcore_axis_name="s")
    f = pl.kernel(
        _sc_body,
        out_type=[
            jax.ShapeDtypeStruct((2, 2, _NP, HID), jnp.float32),
            jax.ShapeDtypeStruct((2, 2, _NPD, HID), jnp.float32),
        ],
        mesh=mesh,
        scratch_types=[
            pltpu.VMEM((_C,), jnp.int32),
            pltpu.VMEM((_C,), jnp.int32),
            pltpu.VMEM((_C,), jnp.int32),
            pltpu.VMEM((_C,), jnp.int32),
            pltpu.VMEM((_C,), jnp.int32),
            pltpu.VMEM((_C, HID), jnp.float32),
            pltpu.VMEM((_C, HID), jnp.float32),
            pltpu.VMEM((_C, HID), jnp.float32),
            pltpu.VMEM((_C, HID), jnp.float32),
            pltpu.VMEM_SHARED((_NP, HID), jnp.float32),
            pltpu.VMEM_SHARED((_NPD, HID), jnp.float32),
            pltpu.SemaphoreType.DMA,
            pltpu.SemaphoreType.DMA,
            pltpu.SemaphoreType.DMA,
        ],
    )
    num_all, den_all = f(h, t, src_all, dst_all)
    den_all = den_all.reshape(2, 2, _NP, 2 * HEADS)
    return num_all, den_all


def kernel(x, proj_W, proj_b, att_src_b, att_dst_b, att_src_u, att_dst_u,
           k_lin_W, k_lin_b, q, lin_W, lin_b,
           edge_index_boundary, edge_index_upper):
    eye = jnp.eye(HEADS, dtype=jnp.float32)

    def amat(a_src, a_dst):
        ms = (a_src[:, :, None] * eye[:, None, :]).reshape(HID, HEADS)
        md = (a_dst[:, :, None] * eye[:, None, :]).reshape(HID, HEADS)
        return jnp.concatenate([ms, md], axis=1)

    Ab = amat(att_src_b, att_dst_b)
    Au = amat(att_src_u, att_dst_u)
    A = jnp.concatenate(
        [Ab[:, :HEADS], Au[:, :HEADS], Ab[:, HEADS:], Au[:, HEADS:],
         jnp.zeros((HID, HID - 4 * HEADS), jnp.float32)], axis=1)
    h, t = _k1(x, proj_W, proj_b[None, :], A)

    num_all, den_all = _edge_stage_sc(
        h, t, edge_index_boundary, edge_index_upper)

    Emat = jnp.repeat(eye, D_HEAD, axis=1)
    acc = _k3(num_all, den_all, k_lin_W, k_lin_b[None, :], Emat)

    sb, su, tb, tu = acc[0], acc[1], acc[2], acc[3]
    k = jnp.stack([tb, tu]) / N
    score = (q[None, :] * k).sum(-1)
    attn = jax.nn.softmax(score, axis=0)
    pooled = attn[0] * sb + attn[1] * su
    out = pooled[None, :] @ lin_W + lin_b[None, :]
    return jax.nn.sigmoid(out)


# speedup vs baseline: 50.1678x; 1.1210x over previous
import functools
import jax
import jax.numpy as jnp
import numpy as np
from jax import lax
from jax.experimental import pallas as pl
from jax.experimental.pallas import tpu as pltpu
from jax.experimental.pallas import tpu_sc as plsc

N = 10000
E = 320000
D_IN = 128
HID = 128
HEADS = 8
D_HEAD = HID // HEADS
OUT = 2

_BLK = 1000
_GRID = N // _BLK

_NW = 32
_C = 64
_EP = ((E + _NW * _C - 1) // (_NW * _C)) * (_NW * _C)
_EPW = _EP // _NW
_NCH = _EPW // _C
_NP = 10240
_RPT = _NP // 16
_NPD = _NP // 8
_RPD = _NPD // 16


def _k1_body(x_ref, W_ref, b_ref, A_ref, h_ref, t_ref):
    h = jnp.dot(x_ref[...], W_ref[...], preferred_element_type=jnp.float32)
    h = h + b_ref[...]
    h_ref[...] = h
    t_ref[...] = jnp.dot(h, A_ref[...], preferred_element_type=jnp.float32)


def _k1(x, W, b2, A):
    return pl.pallas_call(
        _k1_body,
        grid=(_GRID,),
        in_specs=[
            pl.BlockSpec((_BLK, D_IN), lambda i: (i, 0)),
            pl.BlockSpec((D_IN, HID), lambda i: (0, 0)),
            pl.BlockSpec((1, HID), lambda i: (0, 0)),
            pl.BlockSpec((HID, HID), lambda i: (0, 0)),
        ],
        out_specs=[
            pl.BlockSpec((_BLK, HID), lambda i: (i, 0)),
            pl.BlockSpec((_BLK, HID), lambda i: (i, 0)),
        ],
        out_shape=[
            jax.ShapeDtypeStruct((N, HID), jnp.float32),
            jax.ShapeDtypeStruct((N, HID), jnp.float32),
        ],
    )(x, W, b2, A)


def _k3_body(numb_ref, denb_ref, numu_ref, denu_ref, W_ref, bk_ref, E_ref,
             acc_ref):
    i = pl.program_id(0)

    @pl.when(i == 0)
    def _():
        acc_ref[...] = jnp.zeros_like(acc_ref)

    def one(num_ref, den_ref, o):
        num = num_ref[0, 0] + num_ref[0, 1]
        den = (den_ref[0, 0, :, o:o + HEADS]
               + den_ref[0, 1, :, o:o + HEADS])
        dr = jnp.dot(1.0 / (den + 1e-16), E_ref[...],
                     preferred_element_type=jnp.float32)
        o = jnp.maximum(num * dr, 0.0)
        s = jnp.sum(o, axis=0)
        t = jnp.sum(jnp.tanh(
            jnp.dot(o, W_ref[...], preferred_element_type=jnp.float32)
            + bk_ref[...]), axis=0)
        return s, t

    sb, tb = one(numb_ref, denb_ref, 0)
    su, tu = one(numu_ref, denu_ref, HEADS)
    z = jnp.zeros((4, HID), jnp.float32)
    upd = jnp.concatenate([sb[None], su[None], tb[None], tu[None], z], axis=0)
    acc_ref[...] += upd


def _k3(num_all, den_all, kW, kb2, Emat):
    return pl.pallas_call(
        _k3_body,
        grid=(_GRID,),
        in_specs=[
            pl.BlockSpec((1, 2, _BLK, HID), lambda i: (0, 0, i, 0)),
            pl.BlockSpec((1, 2, _BLK, 2 * HEADS), lambda i: (0, 0, i, 0)),
            pl.BlockSpec((1, 2, _BLK, HID), lambda i: (1, 0, i, 0)),
            pl.BlockSpec((1, 2, _BLK, 2 * HEADS), lambda i: (1, 0, i, 0)),
            pl.BlockSpec((HID, HID), lambda i: (0, 0)),
            pl.BlockSpec((1, HID), lambda i: (0, 0)),
            pl.BlockSpec((HEADS, HID), lambda i: (0, 0)),
        ],
        out_specs=pl.BlockSpec((8, HID), lambda i: (0, 0)),
        out_shape=jax.ShapeDtypeStruct((8, HID), jnp.float32),
    )(num_all, den_all, num_all, den_all, kW, kb2, Emat)


def _sc_body(h_hbm, t_hbm, src_all, dst_all,
             num_all, den_all,
             src_i, dst_i, srcC, dstA, dstC, S, Dv, exP, hbuf,
             num_s, den_s, semA, semB, semH, semD, semN):
    cid = lax.axis_index("c")
    sid = lax.axis_index("s")
    wid = sid * 2 + cid
    row0 = sid * _RPT
    zero16i = jnp.zeros((16,), jnp.int32)
    zero16f = jnp.zeros((16,), jnp.float32)

    def zp(i, c):
        exP[i >> 3, pl.ds((i & 7) * 16, 16)] = zero16f
        return c
    lax.fori_loop(0, _C * 8, zp, 0)

    def relbody(rel, cr):
        for j in range(_RPT // _C):
            pltpu.sync_copy(exP, num_s.at[pl.ds(row0 + j * _C, _C)])
        pltpu.sync_copy(exP, den_s.at[pl.ds(sid * _RPD, _C)])
        pltpu.sync_copy(exP.at[pl.ds(0, _RPD - _C)],
                        den_s.at[pl.ds(sid * _RPD + _C, _RPD - _C)])
        plsc.subcore_barrier()

        def chunk(ch, c):
            base = pl.multiple_of(rel * _EP + wid * _EPW + ch * _C, _C)
            @pl.when(ch > 0)
            def _():
                pltpu.make_async_copy(hbuf, num_s.at[dstC], semN).wait()
            iS = pltpu.async_copy(src_all.at[pl.ds(base, _C)], src_i, semA)
            iD = pltpu.async_copy(dst_all.at[pl.ds(base, _C)], dst_i, semB)
            iS.wait()
            iD.wait()
            for j in range(_C // 16):
                sv = src_i[pl.ds(j * 16, 16)]
                dv = dst_i[pl.ds(j * 16, 16)]
                srcC[pl.ds(j * 16, 16)] = jnp.minimum(sv, N - 1)
                dstA[pl.ds(j * 16, 16)] = jnp.minimum(dv, N - 1)
                dc = jnp.minimum(dv, N)
                dstC[pl.ds(j * 16, 16)] = dc
                dst_i[pl.ds(j * 16, 16)] = dc >> 3
            gA = pltpu.async_copy(t_hbm.at[srcC], S, semA)
            gD = pltpu.async_copy(t_hbm.at[dstA], Dv, semB)
            gH = pltpu.async_copy(h_hbm.at[srcC], hbuf, semH)
            gA.wait()
            gD.wait()

            def exloop(j, c2):
                dvv = dstC[pl.ds(j * 16, 16)]
                for l in range(16):
                    e = j * 16 + l
                    v = S[e, pl.ds(0, 16)] + Dv[e, pl.ds(16, 16)]
                    v = jnp.where(v >= 0, v, 0.2 * v)
                    ex = jnp.exp(v)
                    S[e, pl.ds(0, 16)] = ex
                    cb = (dvv[l] & 7) * 16
                    exP[e, pl.ds(cb, 16)] = ex
                return c2
            lax.fori_loop(0, _C // 16, exloop, 0)

            dS = pltpu.async_copy(exP, den_s.at[dst_i], semD, add=True)

            gH.wait()

            def mb(e, c2):
                wv = S[e, pl.ds(0, 16)]
                for hh in range(HEADS):
                    w = lax.gather(
                        wv, (zero16i + (rel * HEADS + hh))[:, None],
                        lax.GatherDimensionNumbers(
                            offset_dims=(), collapsed_slice_dims=(0,),
                            start_index_map=(0,)),
                        slice_sizes=(1,),
                        mode=lax.GatherScatterMode.PROMISE_IN_BOUNDS)
                    hv = hbuf[e, pl.ds(hh * 16, 16)]
                    hbuf[e, pl.ds(hh * 16, 16)] = hv * w
                return c2
            lax.fori_loop(0, _C, mb, 0)

            dS.wait()

            def clr(j, c2):
                dvv = dstC[pl.ds(j * 16, 16)]
                for l in range(16):
                    cb = (dvv[l] & 7) * 16
                    exP[j * 16 + l, pl.ds(cb, 16)] = zero16f
                return c2
            lax.fori_loop(0, _C // 16, clr, 0)

            pltpu.async_copy(hbuf, num_s.at[dstC], semN, add=True)
            return c

        lax.fori_loop(0, _NCH, chunk, 0)
        pltpu.make_async_copy(hbuf, num_s.at[dstC], semN).wait()
        plsc.subcore_barrier()

        pltpu.sync_copy(num_s.at[pl.ds(row0, _RPT)],
                        num_all.at[rel, cid, pl.ds(row0, _RPT)])
        pltpu.sync_copy(den_s.at[pl.ds(sid * _RPD, _RPD)],
                        den_all.at[rel, cid, pl.ds(sid * _RPD, _RPD)])
        plsc.subcore_barrier()
        return cr

    lax.fori_loop(0, 2, relbody, 0)


def _edge_stage_sc(h, t, ei_b, ei_u):
    pad = jnp.full((_EP - E,), N, jnp.int32)
    src_all = jnp.concatenate([ei_b[0], pad, ei_u[0], pad])
    dst_all = jnp.concatenate([ei_b[1], pad, ei_u[1], pad])

    mesh = plsc.VectorSubcoreMesh(core_axis_name="c", subcore_axis_name="s")
    f = pl.kernel(
        _sc_body,
        out_type=[
            jax.ShapeDtypeStruct((2, 2, _NP, HID), jnp.float32),
            jax.ShapeDtypeStruct((2, 2, _NPD, HID), jnp.float32),
        ],
        mesh=mesh,
        scratch_types=[
            pltpu.VMEM((_C,), jnp.int32),
            pltpu.VMEM((_C,), jnp.int32),
            pltpu.VMEM((_C,), jnp.int32),
            pltpu.VMEM((_C,), jnp.int32),
            pltpu.VMEM((_C,), jnp.int32),
            pltpu.VMEM((_C, HID), jnp.float32),
            pltpu.VMEM((_C, HID), jnp.float32),
            pltpu.VMEM((_C, HID), jnp.float32),
            pltpu.VMEM((_C, HID), jnp.float32),
            pltpu.VMEM_SHARED((_NP, HID), jnp.float32),
            pltpu.VMEM_SHARED((_NPD, HID), jnp.float32),
            pltpu.SemaphoreType.DMA,
            pltpu.SemaphoreType.DMA,
            pltpu.SemaphoreType.DMA,
            pltpu.SemaphoreType.DMA,
            pltpu.SemaphoreType.DMA,
        ],
    )
    num_all, den_all = f(h, t, src_all, dst_all)
    den_all = den_all.reshape(2, 2, _NP, 2 * HEADS)
    return num_all, den_all


def kernel(x, proj_W, proj_b, att_src_b, att_dst_b, att_src_u, att_dst_u,
           k_lin_W, k_lin_b, q, lin_W, lin_b,
           edge_index_boundary, edge_index_upper):
    eye = jnp.eye(HEADS, dtype=jnp.float32)

    def amat(a_src, a_dst):
        ms = (a_src[:, :, None] * eye[:, None, :]).reshape(HID, HEADS)
        md = (a_dst[:, :, None] * eye[:, None, :]).reshape(HID, HEADS)
        return jnp.concatenate([ms, md], axis=1)

    Ab = amat(att_src_b, att_dst_b)
    Au = amat(att_src_u, att_dst_u)
    A = jnp.concatenate(
        [Ab[:, :HEADS], Au[:, :HEADS], Ab[:, HEADS:], Au[:, HEADS:],
         jnp.zeros((HID, HID - 4 * HEADS), jnp.float32)], axis=1)
    h, t = _k1(x, proj_W, proj_b[None, :], A)

    num_all, den_all = _edge_stage_sc(
        h, t, edge_index_boundary, edge_index_upper)

    Emat = jnp.repeat(eye, D_HEAD, axis=1)
    acc = _k3(num_all, den_all, k_lin_W, k_lin_b[None, :], Emat)

    sb, su, tb, tu = acc[0], acc[1], acc[2], acc[3]
    k = jnp.stack([tb, tu]) / N
    score = (q[None, :] * k).sum(-1)
    attn = jax.nn.softmax(score, axis=0)
    pooled = attn[0] * sb + attn[1] * su
    out = pooled[None, :] @ lin_W + lin_b[None, :]
    return jax.nn.sigmoid(out)
